# baseline (speedup 1.0000x reference)
import functools

import numpy as np
import jax
import jax.numpy as jnp
from jax import lax
from jax.experimental import pallas as pl
from jax.experimental.pallas import tpu as pltpu
from jax.experimental.pallas import tpu_sc as plsc

N = 16384
F = 128
E = 524288
NG = 128
NPG = 128
H = 256
MLPH = 512
NCLS = 8
EPS = 1e-5
D0 = NPG * (NPG - 1) // 2
D1 = D0 + 3 * H

_BNSCALE = 1.0 / np.sqrt(1.0 + EPS).astype(np.float32)

_IU0, _IU1 = np.triu_indices(NPG, k=1)
_TRIU_NP = (_IU0 * NPG + _IU1).astype(np.int32)


_EB = 16384


def _edge_body(ev_ref, ea_ref, ei_ref, rad_ref, idx_ref):
    v0 = ev_ref[0, :]
    v1 = ev_ref[1, :]
    v2 = ev_ref[2, :]
    r = jnp.sqrt(v0 * v0 + v1 * v1 + v2 * v2)
    rad_ref[...] = ea_ref[...] * jnp.exp(-r) * 0.1
    src = ei_ref[0, :]
    dst = ei_ref[1, :]
    idx_ref[...] = dst * NPG + (src & (NPG - 1))


def _edge_call(edge_vectors, edge_attr_flat, edge_index):
    grid = (E // _EB,)
    return pl.pallas_call(
        _edge_body,
        grid=grid,
        in_specs=[
            pl.BlockSpec((3, _EB), lambda i: (0, i)),
            pl.BlockSpec((_EB,), lambda i: (i,)),
            pl.BlockSpec((2, _EB), lambda i: (0, i)),
        ],
        out_specs=[
            pl.BlockSpec((_EB,), lambda i: (i,)),
            pl.BlockSpec((_EB,), lambda i: (i,)),
        ],
        out_shape=[
            jax.ShapeDtypeStruct((E,), jnp.float32),
            jax.ShapeDtypeStruct((E,), jnp.int32),
        ],
    )(edge_vectors, edge_attr_flat, edge_index)


_ROWS = E // NPG
_RPT = _ROWS // 16
_CH = 32
_SH = 32
_HALF = (NG // 2) * NPG * NPG
_DUMMY = _HALF
_TABPAD = _HALF + 64
_STRIPE = _HALF // 16
_ZB = 2048
_GPT = NG // 32


def _sc_body(fi_hbm, rad_hbm, x_hbm, triu_hbm,
             tab_out, xt_out,
             table, fib, radb, idxc, radc, idx2d, zb, xg, xtb, triub, sem):
    cid = lax.axis_index("c")
    sid = lax.axis_index("s")
    cbase = cid * _HALF

    zeros16 = jnp.zeros((16,), jnp.float32)

    scope_zero = jax.named_scope("sczero")
    scope_zero.__enter__()

    def zb_body(i, _):
        zb[pl.ds(i * 16, 16)] = zeros16
        return 0

    lax.fori_loop(0, _ZB // 16, zb_body, 0)

    def zfill(i, _):
        pltpu.sync_copy(zb, table.at[pl.ds(sid * _STRIPE + i * _ZB, _ZB)])
        return 0

    lax.fori_loop(0, _STRIPE // _ZB, zfill, 0)
    plsc.subcore_barrier()
    scope_zero.__exit__(None, None, None)
    scope_scat = jax.named_scope("scscat")
    scope_scat.__enter__()

    def prefill_dummy(i, _):
        idx2d[i] = jnp.full((NPG,), _DUMMY, jnp.int32)
        return 0

    def prefill_dummy_rows(i, _):
        for k in range(NPG // 16):
            idx2d[i, pl.ds(k * 16, 16)] = jnp.full((16,), _DUMMY, jnp.int32)
            idxc[pl.ds(i * NPG + k * 16, 16)] = jnp.full((16,), _DUMMY,
                                                         jnp.int32)
        return 0

    lax.fori_loop(0, _SH + 1, prefill_dummy_rows, 0)

    def pass_body(h, _):
        r0 = sid * _RPT + h * _SH
        ed_i = pltpu.async_copy(fi_hbm.at[pl.ds(r0, _SH)], fib, sem)
        ed_r = pltpu.async_copy(rad_hbm.at[pl.ds(r0, _SH)], radb, sem)

        def rzero(i, _):
            radc[pl.ds(i * 16, 16)] = jnp.zeros((16,), jnp.float32)
            return 0

        lax.fori_loop(0, (_SH + 1) * NPG // 16, rzero, 0)
        ed_i.wait()
        ed_r.wait()

        def row_body(j, cnt):
            for k in range(NPG // 16):
                v = fib[j, pl.ds(k * 16, 16)]
                rr = radb[j, pl.ds(k * 16, 16)]
                loc = v - cbase
                ok = (loc >= 0) & (loc < _HALF)
                plsc.store_compressed(idxc.at[pl.ds(cnt, 16)], loc, mask=ok)
                plsc.store_compressed(radc.at[pl.ds(cnt, 16)], rr, mask=ok)
                cnt = cnt + jnp.sum(ok.astype(jnp.int32))
            return cnt

        cnt = lax.fori_loop(0, _SH, row_body, jnp.int32(0))
        nrows = (cnt + NPG - 1) // NPG

        def crow(j, _):
            for k in range(NPG // 16):
                idx2d[j, pl.ds(k * 16, 16)] = idxc[pl.ds(j * NPG + k * 16,
                                                         16)]
            return 0

        lax.fori_loop(0, nrows, crow, 0)

        for k in range(_SH + 1):
            @pl.when(k < nrows)
            def _():
                pltpu.async_copy(radc.at[pl.ds(k * NPG, NPG)],
                                 table.at[idx2d.at[k]], sem, add=True)
        for k in range(_SH + 1):
            @pl.when(k < nrows)
            def _():
                pltpu.make_async_copy(radc.at[pl.ds(k * NPG, NPG)],
                                      table.at[idx2d.at[k]], sem).wait()
        return 0

    lax.fori_loop(0, _RPT // _SH, pass_body, 0)
    plsc.subcore_barrier()
    scope_scat.__exit__(None, None, None)
    scope_exp = jax.named_scope("scexp")
    scope_exp.__enter__()
    pltpu.sync_copy(table.at[pl.ds(sid * _STRIPE, _STRIPE)],
                    tab_out.at[pl.ds(cbase + sid * _STRIPE, _STRIPE)])
    scope_exp.__exit__(None, None, None)

    scope_g = jax.named_scope("scgat")
    scope_g.__enter__()
    w = sid * 2 + cid
    pltpu.sync_copy(triu_hbm, triub)

    def gbody(k, _):
        g = w * _GPT + k
        pltpu.sync_copy(x_hbm.at[pl.ds(g * NPG * F, NPG * F)], xg)

        def pbody(p, _):
            ti = triub[pl.ds(p * 16, 16)]
            xtb[pl.ds(p * 16, 16)] = plsc.load_gather(xg, [ti])
            return 0

        lax.fori_loop(0, D0 // 16, pbody, 0)
        pltpu.sync_copy(xtb, xt_out.at[pl.ds(g * D0, D0)])
        return 0

    lax.fori_loop(0, _GPT, gbody, 0)
    scope_g.__exit__(None, None, None)


def _sc_call(flatidx, radial, xflat, triu):
    mesh = plsc.VectorSubcoreMesh(core_axis_name="c", subcore_axis_name="s")
    kern = pl.kernel(
        _sc_body,
        out_type=(
            jax.ShapeDtypeStruct((NG * NPG * NPG,), jnp.float32),
            jax.ShapeDtypeStruct((NG * D0,), jnp.float32),
        ),
        mesh=mesh,
        compiler_params=pltpu.CompilerParams(needs_layout_passes=False),
        scratch_types=[
            pltpu.VMEM_SHARED((_TABPAD,), jnp.float32),
            pltpu.VMEM((_SH, NPG), jnp.int32),
            pltpu.VMEM((_SH, NPG), jnp.float32),
            pltpu.VMEM(((_SH + 1) * NPG,), jnp.int32),
            pltpu.VMEM(((_SH + 1) * NPG,), jnp.float32),
            pltpu.VMEM((_SH + 1, NPG), jnp.int32),
            pltpu.VMEM((_ZB,), jnp.float32),
            pltpu.VMEM((NPG * F,), jnp.float32),
            pltpu.VMEM((D0,), jnp.float32),
            pltpu.VMEM((D0,), jnp.int32),
            pltpu.SemaphoreType.DMA,
        ],
    )
    return kern(flatidx.reshape(_ROWS, NPG), radial.reshape(_ROWS, NPG),
                xflat, triu)


def _graph_body(x_ref, a_ref, wa_ref, ba_ref,
                wm0_ref, wo0_ref, ws0_ref,
                wm1_ref, wo1_ref, ws1_ref,
                wm2_ref, wo2_ref, ws2_ref,
                bnhg_ref, bnhb_ref, out_ref):
    bf = jnp.bfloat16

    def dot(a, b):
        return jnp.dot(a.astype(bf), b.astype(bf),
                       preferred_element_type=jnp.float32)

    xg = x_ref[0]
    ag = a_ref[0]
    h1 = jnp.maximum(dot(xg, wa_ref[...]) + ba_ref[...], 0.0)
    f = h1
    pooled = []
    for wm_ref, wo_ref, ws_ref, fin in (
        (wm0_ref, wo0_ref, ws0_ref, F),
        (wm1_ref, wo1_ref, ws1_ref, H),
        (wm2_ref, wo2_ref, ws2_ref, H),
    ):
        y = dot(f, wm_ref[...])
        agg = dot(ag, y)
        ws = ws_ref[...]
        sc = dot(f, ws[:fin, :]) + dot(xg, ws[fin:, :])
        f = jnp.tanh(dot(agg, wo_ref[...]) + sc)
        pooled.append(jnp.mean(f, axis=0, keepdims=True))
    h = jnp.concatenate(pooled, axis=1)
    out_ref[0] = h * (bnhg_ref[...] * _BNSCALE) + bnhb_ref[...]


def _graph_call(x3, a3, W_a, b_a, Wm0, Wo0, Ws0, Wm1, Wo1, Ws1,
                Wm2, Wo2, Ws2, bnh_g, bnh_b):
    grid = (NG,)
    full = lambda shape: pl.BlockSpec(shape, lambda g: tuple(0 for _ in shape))
    return pl.pallas_call(
        _graph_body,
        grid=grid,
        in_specs=[
            pl.BlockSpec((1, NPG, F), lambda g: (g, 0, 0)),
            pl.BlockSpec((1, NPG, NPG), lambda g: (g, 0, 0)),
            full((F, F)), full((1, F)),
            full((F, H)), full((H, H)), full((F + F, H)),
            full((H, H)), full((H, H)), full((H + F, H)),
            full((H, H)), full((H, H)), full((H + F, H)),
            full((1, 3 * H)), full((1, 3 * H)),
        ],
        out_specs=pl.BlockSpec((1, 1, 3 * H), lambda g: (g, 0, 0)),
        out_shape=jax.ShapeDtypeStruct((NG, 1, 3 * H), jnp.float32),
    )(x3, a3, W_a, b_a, Wm0, Wo0, Ws0, Wm1, Wo1, Ws1, Wm2, Wo2, Ws2,
      bnh_g, bnh_b)


def _head_body(xt_ref, h_ref, bng_ref, bnb_ref,
               w1_ref, b1_ref, g1_ref, bt1_ref,
               w2_ref, b2_ref, g2_ref, bt2_ref,
               w3_ref, b3_ref, g3_ref, bt3_ref,
               w4_ref, b4_ref, out_ref):
    dot = functools.partial(jnp.dot, preferred_element_type=jnp.float32)
    xbn = xt_ref[...] * (bng_ref[...] * _BNSCALE) + bnb_ref[...]
    w1 = w1_ref[...]
    z = dot(xbn, w1[:D0, :]) + dot(h_ref[...], w1[D0:, :]) + b1_ref[...]
    z = jnp.maximum(z * (g1_ref[...] * _BNSCALE) + bt1_ref[...], 0.0)
    z = dot(z, w2_ref[...]) + b2_ref[...]
    z = jnp.maximum(z * (g2_ref[...] * _BNSCALE) + bt2_ref[...], 0.0)
    z = dot(z, w3_ref[...]) + b3_ref[...]
    z = jnp.maximum(z * (g3_ref[...] * _BNSCALE) + bt3_ref[...], 0.0)
    z = dot(z, w4_ref[...]) + b4_ref[...]
    m = jnp.max(z, axis=1, keepdims=True)
    lse = m + jnp.log(jnp.sum(jnp.exp(z - m), axis=1, keepdims=True))
    out_ref[...] = z - lse


def _head_call(xt, h, bn_g, bn_b, W1, b1, g1, bt1, W2, b2, g2, bt2,
               W3, b3, g3, bt3, W4, b4):
    return pl.pallas_call(
        _head_body,
        out_shape=jax.ShapeDtypeStruct((NG, NCLS), jnp.float32),
    )(xt, h, bn_g, bn_b, W1, b1, g1, bt1, W2, b2, g2, bt2,
      W3, b3, g3, bt3, W4, b4)


def kernel(x, edge_vectors, edge_attr, W_a, b_a, W_msg0, W_out0, W_sc0,
           W_msg1, W_out1, W_sc1, W_msg2, W_out2, W_sc2, bn_g, bn_b,
           bnh_g, bnh_b, W1, b1, g1, bt1, W2, b2, g2, bt2, W3, b3, g3, bt3,
           W4, b4, edge_index, batch):
    del batch
    radial, flatidx = _edge_call(edge_vectors, edge_attr[:, 0], edge_index)
    triu = jnp.asarray(_TRIU_NP)
    tabf, xtf = _sc_call(flatidx, radial, x.reshape(-1), triu)
    h = _graph_call(
        x.reshape(NG, NPG, F), tabf.reshape(NG, NPG, NPG),
        W_a, b_a.reshape(1, F),
        W_msg0, W_out0, W_sc0, W_msg1, W_out1, W_sc1, W_msg2, W_out2, W_sc2,
        bnh_g.reshape(1, 3 * H), bnh_b.reshape(1, 3 * H))
    return _head_call(
        xtf.reshape(NG, D0), h.reshape(NG, 3 * H),
        bn_g.reshape(1, D0), bn_b.reshape(1, D0),
        W1, b1.reshape(1, MLPH), g1.reshape(1, MLPH), bt1.reshape(1, MLPH),
        W2, b2.reshape(1, MLPH // 2), g2.reshape(1, MLPH // 2),
        bt2.reshape(1, MLPH // 2),
        W3, b3.reshape(1, MLPH // 2), g3.reshape(1, MLPH // 2),
        bt3.reshape(1, MLPH // 2),
        W4, b4.reshape(1, NCLS))


# speedup vs baseline: 1.5782x; 1.5782x over previous
import functools

import numpy as np
import jax
import jax.numpy as jnp
from jax import lax
from jax.experimental import pallas as pl
from jax.experimental.pallas import tpu as pltpu
from jax.experimental.pallas import tpu_sc as plsc

N = 16384
F = 128
E = 524288
NG = 128
NPG = 128
H = 256
MLPH = 512
NCLS = 8
EPS = 1e-5
D0 = NPG * (NPG - 1) // 2
D1 = D0 + 3 * H

_BNSCALE = 1.0 / np.sqrt(1.0 + EPS).astype(np.float32)

_IU0, _IU1 = np.triu_indices(NPG, k=1)
_TRIU_NP = (_IU0 * NPG + _IU1).astype(np.int32)


_EB = 16384


def _edge_body(ev_ref, ea_ref, ei_ref, rad_ref, idx_ref):
    v0 = ev_ref[0, :]
    v1 = ev_ref[1, :]
    v2 = ev_ref[2, :]
    r = jnp.sqrt(v0 * v0 + v1 * v1 + v2 * v2)
    rad_ref[...] = ea_ref[...] * jnp.exp(-r) * 0.1
    src = ei_ref[0, :]
    dst = ei_ref[1, :]
    idx_ref[...] = dst * NPG + (src & (NPG - 1))


def _edge_call(edge_vectors, edge_attr_flat, edge_index):
    grid = (E // _EB,)
    return pl.pallas_call(
        _edge_body,
        grid=grid,
        in_specs=[
            pl.BlockSpec((3, _EB), lambda i: (0, i)),
            pl.BlockSpec((_EB,), lambda i: (i,)),
            pl.BlockSpec((2, _EB), lambda i: (0, i)),
        ],
        out_specs=[
            pl.BlockSpec((_EB,), lambda i: (i,)),
            pl.BlockSpec((_EB,), lambda i: (i,)),
        ],
        out_shape=[
            jax.ShapeDtypeStruct((E,), jnp.float32),
            jax.ShapeDtypeStruct((E,), jnp.int32),
        ],
    )(edge_vectors, edge_attr_flat, edge_index)


_ROWS = E // NPG
_RPT = _ROWS // 16
_CH = 32
_SH = 32
_HALF = (NG // 2) * NPG * NPG
_DUMMY = _HALF
_TABPAD = _HALF + 64
_STRIPE = _HALF // 16
_ZB = 2048
_GPT = NG // 32


def _sc_body(fi_hbm, rad_hbm, x_hbm, triu_hbm,
             tab_out, xt_out,
             table, fib, radb, idxc, radc, idx2d, zb, xg, xtb, triub, sem):
    cid = lax.axis_index("c")
    sid = lax.axis_index("s")
    cbase = cid * _HALF

    zeros16 = jnp.zeros((16,), jnp.float32)

    scope_zero = jax.named_scope("sczero")
    scope_zero.__enter__()

    def zb_body(i, _):
        zb[pl.ds(i * 16, 16)] = zeros16
        return 0

    lax.fori_loop(0, _ZB // 16, zb_body, 0)

    def zfill(i, _):
        pltpu.sync_copy(zb, table.at[pl.ds(sid * _STRIPE + i * _ZB, _ZB)])
        return 0

    lax.fori_loop(0, _STRIPE // _ZB, zfill, 0)
    plsc.subcore_barrier()
    scope_zero.__exit__(None, None, None)
    scope_scat = jax.named_scope("scscat")
    scope_scat.__enter__()

    def prefill_dummy(i, _):
        idx2d[i] = jnp.full((NPG,), _DUMMY, jnp.int32)
        return 0

    def prefill_dummy_rows(i, _):
        for k in range(NPG // 16):
            idx2d[i, pl.ds(k * 16, 16)] = jnp.full((16,), _DUMMY, jnp.int32)
            idxc[pl.ds(i * NPG + k * 16, 16)] = jnp.full((16,), _DUMMY,
                                                         jnp.int32)
        return 0

    lax.fori_loop(0, _SH + 1, prefill_dummy_rows, 0)

    def pass_body(h, _):
        r0 = sid * _RPT + h * _SH
        ed_i = pltpu.async_copy(fi_hbm.at[pl.ds(r0, _SH)], fib, sem)
        ed_r = pltpu.async_copy(rad_hbm.at[pl.ds(r0, _SH)], radb, sem)

        def rzero(i, _):
            radc[pl.ds(i * 16, 16)] = jnp.zeros((16,), jnp.float32)
            return 0

        lax.fori_loop(0, (_SH + 1) * NPG // 16, rzero, 0)
        ed_i.wait()
        ed_r.wait()

        def row_body(j, cnt):
            for k in range(NPG // 16):
                v = fib[j, pl.ds(k * 16, 16)]
                rr = radb[j, pl.ds(k * 16, 16)]
                loc = v - cbase
                ok = (loc >= 0) & (loc < _HALF)
                plsc.store_compressed(idxc.at[pl.ds(cnt, 16)], loc, mask=ok)
                plsc.store_compressed(radc.at[pl.ds(cnt, 16)], rr, mask=ok)
                cnt = cnt + jnp.sum(ok.astype(jnp.int32))
            return cnt

        cnt = lax.fori_loop(0, _SH, row_body, jnp.int32(0))
        nrows = (cnt + NPG - 1) // NPG

        def crow(j, _):
            for k in range(NPG // 16):
                idx2d[j, pl.ds(k * 16, 16)] = idxc[pl.ds(j * NPG + k * 16,
                                                         16)]
            return 0

        lax.fori_loop(0, nrows, crow, 0)

        for k in range(_SH + 1):
            @pl.when(k < nrows)
            def _():
                pltpu.async_copy(radc.at[pl.ds(k * NPG, NPG)],
                                 table.at[idx2d.at[k]], sem, add=True)
        for k in range(_SH + 1):
            @pl.when(k < nrows)
            def _():
                pltpu.make_async_copy(radc.at[pl.ds(k * NPG, NPG)],
                                      table.at[idx2d.at[k]], sem).wait()
        return 0

    lax.fori_loop(0, _RPT // _SH, pass_body, 0)
    plsc.subcore_barrier()
    scope_scat.__exit__(None, None, None)
    scope_exp = jax.named_scope("scexp")
    scope_exp.__enter__()
    pltpu.sync_copy(table.at[pl.ds(sid * _STRIPE, _STRIPE)],
                    tab_out.at[pl.ds(cbase + sid * _STRIPE, _STRIPE)])
    scope_exp.__exit__(None, None, None)

    scope_g = jax.named_scope("scgat")
    scope_g.__enter__()
    w = sid * 2 + cid
    pltpu.sync_copy(triu_hbm, triub)

    def gbody(k, _):
        g = w * _GPT + k
        pltpu.sync_copy(x_hbm.at[pl.ds(g * NPG * F, NPG * F)], xg)

        def pbody(p, _):
            ti = triub[pl.ds(p * 16, 16)]
            xtb[pl.ds(p * 16, 16)] = plsc.load_gather(xg, [ti])
            return 0

        lax.fori_loop(0, D0 // 16, pbody, 0)
        pltpu.sync_copy(xtb, xt_out.at[pl.ds(g * D0, D0)])
        return 0

    lax.fori_loop(0, _GPT, gbody, 0)
    scope_g.__exit__(None, None, None)


def _sc_call(flatidx, radial, xflat, triu):
    mesh = plsc.VectorSubcoreMesh(core_axis_name="c", subcore_axis_name="s")
    kern = pl.kernel(
        _sc_body,
        out_type=(
            jax.ShapeDtypeStruct((NG * NPG * NPG,), jnp.float32),
            jax.ShapeDtypeStruct((NG * D0,), jnp.float32),
        ),
        mesh=mesh,
        compiler_params=pltpu.CompilerParams(needs_layout_passes=False),
        scratch_types=[
            pltpu.VMEM_SHARED((_TABPAD,), jnp.float32),
            pltpu.VMEM((_SH, NPG), jnp.int32),
            pltpu.VMEM((_SH, NPG), jnp.float32),
            pltpu.VMEM(((_SH + 1) * NPG,), jnp.int32),
            pltpu.VMEM(((_SH + 1) * NPG,), jnp.float32),
            pltpu.VMEM((_SH + 1, NPG), jnp.int32),
            pltpu.VMEM((_ZB,), jnp.float32),
            pltpu.VMEM((NPG * F,), jnp.float32),
            pltpu.VMEM((D0,), jnp.float32),
            pltpu.VMEM((D0,), jnp.int32),
            pltpu.SemaphoreType.DMA,
        ],
    )
    return kern(flatidx.reshape(_ROWS, NPG), radial.reshape(_ROWS, NPG),
                xflat, triu)


_GB = 4


def _graph_body(x_ref, a_ref, wa_ref, ba_ref,
                wm0_ref, wo0_ref, ws0_ref,
                wm1_ref, wo1_ref, ws1_ref,
                wm2_ref, wo2_ref, ws2_ref,
                bnhg_ref, bnhb_ref, out_ref):
    bf = jnp.bfloat16

    def dot(a, b):
        return jnp.dot(a.astype(bf), b.astype(bf),
                       preferred_element_type=jnp.float32)

    xg = x_ref[...].reshape(_GB * NPG, F)
    h1 = jnp.maximum(dot(xg, wa_ref[...]) + ba_ref[...], 0.0)
    f = h1
    pooled = []
    for wm_ref, wo_ref, ws_ref, fin in (
        (wm0_ref, wo0_ref, ws0_ref, F),
        (wm1_ref, wo1_ref, ws1_ref, H),
        (wm2_ref, wo2_ref, ws2_ref, H),
    ):
        y = dot(f, wm_ref[...])
        aggs = [dot(a_ref[gg], y[gg * NPG:(gg + 1) * NPG, :])
                for gg in range(_GB)]
        agg = jnp.concatenate(aggs, axis=0)
        ws = ws_ref[...]
        sc = dot(f, ws[:fin, :]) + dot(xg, ws[fin:, :])
        f = jnp.tanh(dot(agg, wo_ref[...]) + sc)
        pooled.append(f)
    bnhg = bnhg_ref[...] * _BNSCALE
    bnhb = bnhb_ref[...]
    for gg in range(_GB):
        hg = jnp.concatenate(
            [jnp.mean(p[gg * NPG:(gg + 1) * NPG, :], axis=0, keepdims=True)
             for p in pooled], axis=1)
        out_ref[gg] = hg * bnhg + bnhb


def _graph_call(x3, a3, W_a, b_a, Wm0, Wo0, Ws0, Wm1, Wo1, Ws1,
                Wm2, Wo2, Ws2, bnh_g, bnh_b):
    grid = (NG // _GB,)
    full = lambda shape: pl.BlockSpec(shape, lambda g: tuple(0 for _ in shape))
    return pl.pallas_call(
        _graph_body,
        grid=grid,
        in_specs=[
            pl.BlockSpec((_GB, NPG, F), lambda g: (g, 0, 0)),
            pl.BlockSpec((_GB, NPG, NPG), lambda g: (g, 0, 0)),
            full((F, F)), full((1, F)),
            full((F, H)), full((H, H)), full((F + F, H)),
            full((H, H)), full((H, H)), full((H + F, H)),
            full((H, H)), full((H, H)), full((H + F, H)),
            full((1, 3 * H)), full((1, 3 * H)),
        ],
        out_specs=pl.BlockSpec((_GB, 1, 3 * H), lambda g: (g, 0, 0)),
        out_shape=jax.ShapeDtypeStruct((NG, 1, 3 * H), jnp.float32),
    )(x3, a3, W_a, b_a, Wm0, Wo0, Ws0, Wm1, Wo1, Ws1, Wm2, Wo2, Ws2,
      bnh_g, bnh_b)


def _head_body(xt_ref, h_ref, bng_ref, bnb_ref,
               w1_ref, b1_ref, g1_ref, bt1_ref,
               w2_ref, b2_ref, g2_ref, bt2_ref,
               w3_ref, b3_ref, g3_ref, bt3_ref,
               w4_ref, b4_ref, out_ref):
    dot = functools.partial(jnp.dot, preferred_element_type=jnp.float32)
    xbn = xt_ref[...] * (bng_ref[...] * _BNSCALE) + bnb_ref[...]
    w1 = w1_ref[...]
    z = dot(xbn, w1[:D0, :]) + dot(h_ref[...], w1[D0:, :]) + b1_ref[...]
    z = jnp.maximum(z * (g1_ref[...] * _BNSCALE) + bt1_ref[...], 0.0)
    z = dot(z, w2_ref[...]) + b2_ref[...]
    z = jnp.maximum(z * (g2_ref[...] * _BNSCALE) + bt2_ref[...], 0.0)
    z = dot(z, w3_ref[...]) + b3_ref[...]
    z = jnp.maximum(z * (g3_ref[...] * _BNSCALE) + bt3_ref[...], 0.0)
    z = dot(z, w4_ref[...]) + b4_ref[...]
    m = jnp.max(z, axis=1, keepdims=True)
    lse = m + jnp.log(jnp.sum(jnp.exp(z - m), axis=1, keepdims=True))
    out_ref[...] = z - lse


def _head_call(xt, h, bn_g, bn_b, W1, b1, g1, bt1, W2, b2, g2, bt2,
               W3, b3, g3, bt3, W4, b4):
    return pl.pallas_call(
        _head_body,
        out_shape=jax.ShapeDtypeStruct((NG, NCLS), jnp.float32),
    )(xt, h, bn_g, bn_b, W1, b1, g1, bt1, W2, b2, g2, bt2,
      W3, b3, g3, bt3, W4, b4)


def kernel(x, edge_vectors, edge_attr, W_a, b_a, W_msg0, W_out0, W_sc0,
           W_msg1, W_out1, W_sc1, W_msg2, W_out2, W_sc2, bn_g, bn_b,
           bnh_g, bnh_b, W1, b1, g1, bt1, W2, b2, g2, bt2, W3, b3, g3, bt3,
           W4, b4, edge_index, batch):
    del batch
    radial, flatidx = _edge_call(edge_vectors, edge_attr[:, 0], edge_index)
    triu = jnp.asarray(_TRIU_NP)
    tabf, xtf = _sc_call(flatidx, radial, x.reshape(-1), triu)
    h = _graph_call(
        x.reshape(NG, NPG, F), tabf.reshape(NG, NPG, NPG),
        W_a, b_a.reshape(1, F),
        W_msg0, W_out0, W_sc0, W_msg1, W_out1, W_sc1, W_msg2, W_out2, W_sc2,
        bnh_g.reshape(1, 3 * H), bnh_b.reshape(1, 3 * H))
    return _head_call(
        xtf.reshape(NG, D0), h.reshape(NG, 3 * H),
        bn_g.reshape(1, D0), bn_b.reshape(1, D0),
        W1, b1.reshape(1, MLPH), g1.reshape(1, MLPH), bt1.reshape(1, MLPH),
        W2, b2.reshape(1, MLPH // 2), g2.reshape(1, MLPH // 2),
        bt2.reshape(1, MLPH // 2),
        W3, b3.reshape(1, MLPH // 2), g3.reshape(1, MLPH // 2),
        bt3.reshape(1, MLPH // 2),
        W4, b4.reshape(1, NCLS))
